# Initial kernel scaffold; baseline (speedup 1.0000x reference)
import jax
import jax.numpy as jnp
from jax.experimental import pallas as pl


def kernel(x, embedding):
    raise NotImplementedError("write your pallas kernel here")


# speedup vs baseline: 2.9691x; 2.9691x over previous
import functools

import jax
import jax.numpy as jnp
from jax import lax
from jax.experimental import pallas as pl
from jax.experimental.pallas import tpu as pltpu
from jax.experimental.pallas import tpu_sc as plsc

BATCH = 4096
HIST = 50
EMBED = 128

NC = 2
NS = 16
NW = NC * NS

N_ROWS = BATCH * HIST
PER_W = N_ROWS // NW
CHUNK = 128
N_CHUNKS = PER_W // CHUNK


def _body(idx_hbm, table_hbm, out_hbm, idx_v, rows_v, gsem):
    wid = lax.axis_index("s") * NC + lax.axis_index("c")
    base = wid * PER_W
    pltpu.sync_copy(idx_hbm.at[wid], idx_v)

    def chunk(j, carry):
        pltpu.async_copy(table_hbm.at[idx_v.at[j]], rows_v, gsem).wait()
        pltpu.sync_copy(rows_v, out_hbm.at[pl.ds(base + j * CHUNK, CHUNK)])
        return carry

    lax.fori_loop(0, N_CHUNKS, chunk, 0)


@jax.jit
def _embed(idx, table):
    mesh = plsc.VectorSubcoreMesh(core_axis_name="c", subcore_axis_name="s")
    run = functools.partial(
        pl.kernel,
        out_type=jax.ShapeDtypeStruct((N_ROWS, EMBED), jnp.float32),
        mesh=mesh,
        scratch_types=[
            pltpu.VMEM((N_CHUNKS, CHUNK), jnp.int32),
            pltpu.VMEM((CHUNK, EMBED), jnp.float32),
            pltpu.SemaphoreType.DMA,
        ],
    )(_body)
    return run(idx, table)


def kernel(x, embedding):
    idx = jnp.asarray(x, jnp.int32).reshape(NW, N_CHUNKS, CHUNK)
    out = _embed(idx, embedding)
    return out.reshape(BATCH, HIST, EMBED)


# speedup vs baseline: 3.3308x; 1.1218x over previous
import functools

import jax
import jax.numpy as jnp
from jax import lax
from jax.experimental import pallas as pl
from jax.experimental.pallas import tpu as pltpu
from jax.experimental.pallas import tpu_sc as plsc

BATCH = 4096
HIST = 50
EMBED = 128

NC = 2
NS = 16
NW = NC * NS

N_ROWS = BATCH * HIST
PER_W = N_ROWS // NW
CHUNK = 128
N_CHUNKS = PER_W // CHUNK


def _body(idx_hbm, table_hbm, out_hbm, idx_v, rows_v, gsem, wsem):
    wid = lax.axis_index("s") * NC + lax.axis_index("c")
    base = wid * PER_W
    pltpu.sync_copy(idx_hbm.at[wid], idx_v)

    def start_gather(j, buf):
        pltpu.async_copy(table_hbm.at[idx_v.at[j]], rows_v.at[buf], gsem)

    def wait_gather(j, buf):
        pltpu.make_async_copy(table_hbm.at[idx_v.at[j]], rows_v.at[buf], gsem).wait()

    def start_write(j, buf):
        pltpu.async_copy(
            rows_v.at[buf], out_hbm.at[pl.ds(base + j * CHUNK, CHUNK)], wsem)

    def wait_write(buf):
        pltpu.make_async_copy(
            rows_v.at[buf], out_hbm.at[pl.ds(base, CHUNK)], wsem).wait()

    start_gather(0, 0)

    def step(j, carry):
        cur = j % 2
        nxt = 1 - cur
        @pl.when(j >= 1)
        def _():
            wait_write(nxt)

        @pl.when(j + 1 < N_CHUNKS)
        def _():
            start_gather(j + 1, nxt)

        wait_gather(j, cur)
        start_write(j, cur)
        return carry

    lax.fori_loop(0, N_CHUNKS, step, 0)
    wait_write((N_CHUNKS - 1) % 2)


@jax.jit
def _embed(idx, table):
    mesh = plsc.VectorSubcoreMesh(core_axis_name="c", subcore_axis_name="s")
    run = functools.partial(
        pl.kernel,
        out_type=jax.ShapeDtypeStruct((N_ROWS, EMBED), jnp.float32),
        mesh=mesh,
        scratch_types=[
            pltpu.VMEM((N_CHUNKS, CHUNK), jnp.int32),
            pltpu.VMEM((2, CHUNK, EMBED), jnp.float32),
            pltpu.SemaphoreType.DMA,
            pltpu.SemaphoreType.DMA,
        ],
    )(_body)
    return run(idx, table)


def kernel(x, embedding):
    idx = jnp.asarray(x, jnp.int32).reshape(NW, N_CHUNKS, CHUNK)
    out = _embed(idx, embedding)
    return out.reshape(BATCH, HIST, EMBED)


# speedup vs baseline: 3.3395x; 1.0026x over previous
import functools

import jax
import jax.numpy as jnp
from jax import lax
from jax.experimental import pallas as pl
from jax.experimental.pallas import tpu as pltpu
from jax.experimental.pallas import tpu_sc as plsc

BATCH = 4096
HIST = 50
EMBED = 128

NC = 2
NS = 16
NW = NC * NS

N_ROWS = BATCH * HIST
PER_W = N_ROWS // NW
CHUNK = 128
N_CHUNKS = PER_W // CHUNK


def _body(idx_hbm, table_hbm, out_hbm, idx_v, rows_v, gsem, wsem):
    wid = lax.axis_index("s") * NC + lax.axis_index("c")
    base = wid * PER_W
    pltpu.sync_copy(idx_hbm.at[wid], idx_v)

    def start_gather(j, buf):
        pltpu.async_copy(table_hbm.at[idx_v.at[j]], rows_v.at[buf], gsem)

    def wait_gather(j, buf):
        pltpu.make_async_copy(table_hbm.at[idx_v.at[j]], rows_v.at[buf], gsem).wait()

    def start_write(j, buf):
        pltpu.async_copy(
            rows_v.at[buf], out_hbm.at[pl.ds(base + j * CHUNK, CHUNK)], wsem)

    def wait_write(buf):
        pltpu.make_async_copy(
            rows_v.at[buf], out_hbm.at[pl.ds(base, CHUNK)], wsem).wait()

    start_gather(0, 0)

    def step(j, carry):
        cur = j % 2
        nxt = 1 - cur
        @pl.when(j >= 1)
        def _():
            wait_write(nxt)

        @pl.when(j + 1 < N_CHUNKS)
        def _():
            start_gather(j + 1, nxt)

        wait_gather(j, cur)
        start_write(j, cur)
        return carry

    lax.fori_loop(0, N_CHUNKS, step, 0)
    wait_write((N_CHUNKS - 1) % 2)


@jax.jit
def _embed(idx, table):
    mesh = plsc.VectorSubcoreMesh(core_axis_name="c", subcore_axis_name="s")
    run = functools.partial(
        pl.kernel,
        out_type=jax.ShapeDtypeStruct((N_ROWS, EMBED), jnp.float32),
        mesh=mesh,
        scratch_types=[
            pltpu.VMEM((N_CHUNKS, CHUNK), jnp.int32),
            pltpu.VMEM((2, CHUNK, EMBED), jnp.float32),
            pltpu.SemaphoreType.DMA,
            pltpu.SemaphoreType.DMA,
        ],
        compiler_params=pltpu.CompilerParams(use_tc_tiling_on_sc=True),
    )(_body)
    return run(idx, table)


def kernel(x, embedding):
    idx = jnp.asarray(x, jnp.int32).reshape(NW, N_CHUNKS, CHUNK)
    out = _embed(idx, embedding)
    return out.reshape(BATCH, HIST, EMBED)


# speedup vs baseline: 10.3794x; 3.1080x over previous
import functools

import jax
import jax.numpy as jnp
from jax import lax
from jax.experimental import pallas as pl
from jax.experimental.pallas import tpu as pltpu
from jax.experimental.pallas import tpu_sc as plsc

BATCH = 4096
HIST = 50
EMBED = 128

NC = 2
NS = 16
NW = NC * NS

N_ROWS = BATCH * HIST
PER_W = N_ROWS // NW
CHUNK = 128
N_CHUNKS = PER_W // CHUNK


def _body(idx_hbm, table_hbm, out_hbm, idx_v, rows_v, gsem, wsem):
    wid = lax.axis_index("s") * NC + lax.axis_index("c")
    base = wid * PER_W
    pltpu.sync_copy(idx_hbm.at[pl.ds(base, PER_W)], idx_v)

    def start_gather(j, buf):
        pltpu.async_copy(
            table_hbm.at[idx_v.at[pl.ds(j * CHUNK, CHUNK)]], rows_v.at[buf], gsem)

    def wait_gather(j, buf):
        pltpu.make_async_copy(
            table_hbm.at[idx_v.at[pl.ds(j * CHUNK, CHUNK)]], rows_v.at[buf], gsem).wait()

    def start_write(j, buf):
        pltpu.async_copy(
            rows_v.at[buf], out_hbm.at[pl.ds(base + j * CHUNK, CHUNK)], wsem)

    def wait_write(buf):
        pltpu.make_async_copy(
            rows_v.at[buf], out_hbm.at[pl.ds(base, CHUNK)], wsem).wait()

    start_gather(0, 0)

    def step(j, carry):
        cur = j % 2
        nxt = 1 - cur
        @pl.when(j >= 1)
        def _():
            wait_write(nxt)

        @pl.when(j + 1 < N_CHUNKS)
        def _():
            start_gather(j + 1, nxt)

        wait_gather(j, cur)
        start_write(j, cur)
        return carry

    lax.fori_loop(0, N_CHUNKS, step, 0)
    wait_write((N_CHUNKS - 1) % 2)


@jax.jit
def _embed(idx, table):
    mesh = plsc.VectorSubcoreMesh(core_axis_name="c", subcore_axis_name="s")
    run = functools.partial(
        pl.kernel,
        out_type=jax.ShapeDtypeStruct((N_ROWS, EMBED), jnp.float32),
        mesh=mesh,
        scratch_types=[
            pltpu.VMEM((PER_W,), jnp.int32),
            pltpu.VMEM((2, CHUNK, EMBED), jnp.float32),
            pltpu.SemaphoreType.DMA,
            pltpu.SemaphoreType.DMA,
        ],
    )(_body)
    return run(idx, table)


def kernel(x, embedding):
    idx = jnp.swapaxes(jnp.asarray(x, jnp.int32), 0, 1).reshape(N_ROWS)
    out = _embed(idx, embedding)
    return out.reshape(HIST, BATCH, EMBED).swapaxes(0, 1)


# speedup vs baseline: 10.5956x; 1.0208x over previous
import functools

import jax
import jax.numpy as jnp
from jax import lax
from jax.experimental import pallas as pl
from jax.experimental.pallas import tpu as pltpu
from jax.experimental.pallas import tpu_sc as plsc

BATCH = 4096
HIST = 50
EMBED = 128

NC = 2
NS = 16
NW = NC * NS

N_ROWS = BATCH * HIST
PER_W = N_ROWS // NW
CHUNK = 128
N_CHUNKS = PER_W // CHUNK
NBUF = 4


def _body(idx_hbm, table_hbm, out_hbm, idx_v, rows_v, gsem, wsem):
    wid = lax.axis_index("s") * NC + lax.axis_index("c")
    base = wid * PER_W
    pltpu.sync_copy(idx_hbm.at[pl.ds(base, PER_W)], idx_v)

    def start_gather(j, buf):
        pltpu.async_copy(
            table_hbm.at[idx_v.at[pl.ds(j * CHUNK, CHUNK)]], rows_v.at[buf], gsem)

    def wait_gather(j, buf):
        pltpu.make_async_copy(
            table_hbm.at[idx_v.at[pl.ds(j * CHUNK, CHUNK)]], rows_v.at[buf], gsem).wait()

    def start_write(j, buf):
        pltpu.async_copy(
            rows_v.at[buf], out_hbm.at[pl.ds(base + j * CHUNK, CHUNK)], wsem)

    def wait_write(buf):
        pltpu.make_async_copy(
            rows_v.at[buf], out_hbm.at[pl.ds(base, CHUNK)], wsem).wait()

    for j in range(NBUF - 1):
        start_gather(j, j)

    def step(j, carry):
        @pl.when(j >= 1)
        def _():
            wait_write(0)

        @pl.when(j + NBUF - 1 < N_CHUNKS)
        def _():
            start_gather(j + NBUF - 1, (j + NBUF - 1) % NBUF)

        wait_gather(j, j % NBUF)
        start_write(j, j % NBUF)
        return carry

    lax.fori_loop(0, N_CHUNKS, step, 0)
    wait_write(0)


@jax.jit
def _embed(idx, table):
    mesh = plsc.VectorSubcoreMesh(core_axis_name="c", subcore_axis_name="s")
    run = functools.partial(
        pl.kernel,
        out_type=jax.ShapeDtypeStruct((N_ROWS, EMBED), jnp.float32),
        mesh=mesh,
        scratch_types=[
            pltpu.VMEM((PER_W,), jnp.int32),
            pltpu.VMEM((NBUF, CHUNK, EMBED), jnp.float32),
            pltpu.SemaphoreType.DMA,
            pltpu.SemaphoreType.DMA,
        ],
    )(_body)
    return run(idx, table)


def kernel(x, embedding):
    idx = jnp.swapaxes(jnp.asarray(x, jnp.int32), 0, 1).reshape(N_ROWS)
    out = _embed(idx, embedding)
    return out.reshape(HIST, BATCH, EMBED).swapaxes(0, 1)


# speedup vs baseline: 10.6204x; 1.0023x over previous
import functools

import jax
import jax.numpy as jnp
from jax import lax
from jax.experimental import pallas as pl
from jax.experimental.pallas import tpu as pltpu
from jax.experimental.pallas import tpu_sc as plsc

BATCH = 4096
HIST = 50
EMBED = 128

NC = 2
NS = 16
NW = NC * NS

N_ROWS = BATCH * HIST
PER_W = N_ROWS // NW
CHUNK = 128
N_CHUNKS = PER_W // CHUNK
NBUF = 6
PRIME = 3


def _body(idx_hbm, table_hbm, out_hbm, idx_v, rows_v, gsem, wsem):
    wid = lax.axis_index("s") * NC + lax.axis_index("c")
    base = wid * PER_W
    pltpu.sync_copy(idx_hbm.at[pl.ds(base, PER_W)], idx_v)

    def start_gather(j, buf):
        pltpu.async_copy(
            table_hbm.at[idx_v.at[pl.ds(j * CHUNK, CHUNK)]], rows_v.at[buf], gsem)

    def wait_gather(j, buf):
        pltpu.make_async_copy(
            table_hbm.at[idx_v.at[pl.ds(j * CHUNK, CHUNK)]], rows_v.at[buf], gsem).wait()

    def start_write(j, buf):
        pltpu.async_copy(
            rows_v.at[buf], out_hbm.at[pl.ds(base + j * CHUNK, CHUNK)], wsem)

    def wait_write(buf):
        pltpu.make_async_copy(
            rows_v.at[buf], out_hbm.at[pl.ds(base, CHUNK)], wsem).wait()

    for j in range(PRIME):
        start_gather(j, j)

    def step(j, carry):
        @pl.when(j >= NBUF - PRIME)
        def _():
            wait_write(0)

        @pl.when(j + PRIME < N_CHUNKS)
        def _():
            start_gather(j + PRIME, (j + PRIME) % NBUF)

        wait_gather(j, j % NBUF)
        start_write(j, j % NBUF)
        return carry

    lax.fori_loop(0, N_CHUNKS, step, 0)
    for _ in range(NBUF - PRIME):
        wait_write(0)


@jax.jit
def _embed(idx, table):
    mesh = plsc.VectorSubcoreMesh(core_axis_name="c", subcore_axis_name="s")
    run = functools.partial(
        pl.kernel,
        out_type=jax.ShapeDtypeStruct((N_ROWS, EMBED), jnp.float32),
        mesh=mesh,
        scratch_types=[
            pltpu.VMEM((PER_W,), jnp.int32),
            pltpu.VMEM((NBUF, CHUNK, EMBED), jnp.float32),
            pltpu.SemaphoreType.DMA,
            pltpu.SemaphoreType.DMA,
        ],
    )(_body)
    return run(idx, table)


def kernel(x, embedding):
    idx = jnp.swapaxes(jnp.asarray(x, jnp.int32), 0, 1).reshape(N_ROWS)
    out = _embed(idx, embedding)
    return out.reshape(HIST, BATCH, EMBED).swapaxes(0, 1)


# speedup vs baseline: 10.7298x; 1.0103x over previous
import functools

import jax
import jax.numpy as jnp
from jax import lax
from jax.experimental import pallas as pl
from jax.experimental.pallas import tpu as pltpu
from jax.experimental.pallas import tpu_sc as plsc

BATCH = 4096
HIST = 50
EMBED = 128

NC = 2
NS = 16
NW = NC * NS

N_ROWS = BATCH * HIST
CHUNK = BATCH // NW
NBUF = 6
PRIME = 3


def _body(xt_hbm, table_hbm, out_hbm, idx_v, rows_v, gsem, wsem):
    wid = lax.axis_index("s") * NC + lax.axis_index("c")
    bbase = wid * CHUNK
    pltpu.sync_copy(xt_hbm.at[:, pl.ds(bbase, CHUNK)], idx_v)

    def start_gather(t, buf):
        pltpu.async_copy(table_hbm.at[idx_v.at[t]], rows_v.at[buf], gsem)

    def wait_gather(t, buf):
        pltpu.make_async_copy(table_hbm.at[idx_v.at[t]], rows_v.at[buf], gsem).wait()

    def start_write(t, buf):
        pltpu.async_copy(
            rows_v.at[buf], out_hbm.at[pl.ds(t * BATCH + bbase, CHUNK)], wsem)

    def wait_write():
        pltpu.make_async_copy(
            rows_v.at[0], out_hbm.at[pl.ds(bbase, CHUNK)], wsem).wait()

    for t in range(PRIME):
        start_gather(t, t)

    def step(t, carry):
        @pl.when(t >= NBUF - PRIME)
        def _():
            wait_write()

        @pl.when(t + PRIME < HIST)
        def _():
            start_gather(t + PRIME, (t + PRIME) % NBUF)

        wait_gather(t, t % NBUF)
        start_write(t, t % NBUF)
        return carry

    lax.fori_loop(0, HIST, step, 0)
    for _ in range(NBUF - PRIME):
        wait_write()


@jax.jit
def _embed(xt, table):
    mesh = plsc.VectorSubcoreMesh(core_axis_name="c", subcore_axis_name="s")
    run = functools.partial(
        pl.kernel,
        out_type=jax.ShapeDtypeStruct((N_ROWS, EMBED), jnp.float32),
        mesh=mesh,
        scratch_types=[
            pltpu.VMEM((HIST, CHUNK), jnp.int32),
            pltpu.VMEM((NBUF, CHUNK, EMBED), jnp.float32),
            pltpu.SemaphoreType.DMA,
            pltpu.SemaphoreType.DMA,
        ],
        compiler_params=pltpu.CompilerParams(use_tc_tiling_on_sc=True),
    )(_body)
    return run(xt, table)


def kernel(x, embedding):
    xt = jnp.swapaxes(jnp.asarray(x, jnp.int32), 0, 1)
    out = _embed(xt, embedding)
    return out.reshape(HIST, BATCH, EMBED).swapaxes(0, 1)
